# baseline (speedup 1.0000x reference)
import jax
import jax.numpy as jnp
import numpy as np
from jax.experimental import pallas as pl
from jax.experimental.pallas import tpu as pltpu

S = 16
B = 32
V = 100000
VB = 3200
VPAD = 102400
NJ = VPAD // VB
VB2 = 3200
NJ2 = (V + VB2 - 1) // VB2

_TINY = np.float32(np.finfo(np.float32).tiny)
_ROT = (13, 15, 26, 6, 17, 29, 16, 24)
_KS = (np.uint32(0), np.uint32(1), np.uint32(0x1BD11BDB))


def _threefry_bits(cnt):
    x0 = jnp.zeros_like(cnt)
    x1 = cnt + np.uint32(1)
    for blk in range(5):
        rots = _ROT[0:4] if blk % 2 == 0 else _ROT[4:8]
        for r in rots:
            x0 = x0 + x1
            x1 = (x1 << np.uint32(r)) | (x1 >> np.uint32(32 - r))
            x1 = x1 ^ x0
        x0 = x0 + _KS[(blk + 1) % 3]
        x1 = x1 + _KS[(blk + 2) % 3] + np.uint32(blk + 1)
    return x0 ^ x1


def _gumbel(cnt):
    bits = _threefry_bits(cnt)
    fb = jax.lax.bitcast_convert_type(
        (bits >> np.uint32(9)) | np.uint32(0x3F800000), jnp.float32)
    u = jnp.maximum(_TINY, fb - np.float32(1.0))
    return -jnp.log(-jnp.log(u))


def _sample_kernel(x_ref, out_ref, vmax_ref, vidx_ref):
    b = pl.program_id(0)
    j = pl.program_id(1)

    @pl.when(j == 0)
    def _():
        vmax_ref[...] = jnp.full((S, VB), -jnp.inf, jnp.float32)
        vidx_ref[...] = jnp.zeros((S, VB), jnp.int32)

    row = jax.lax.broadcasted_iota(jnp.int32, (S, VB), 0)
    col = jax.lax.broadcasted_iota(jnp.int32, (S, VB), 1) + j * VB
    cnt = (row * (B * V) + b * V + col).astype(jnp.uint32)
    val = _gumbel(cnt) + x_ref[0]

    sel = val > vmax_ref[...]
    vidx_ref[...] = jnp.where(sel, col, vidx_ref[...])
    vmax_ref[...] = jnp.where(sel, val, vmax_ref[...])

    @pl.when(j == NJ - 1)
    def _():
        vm = vmax_ref[...]
        m = jnp.max(vm, axis=1, keepdims=True)
        cand = jnp.where(vm == m, vidx_ref[...], jnp.int32(2**31 - 1))
        idx = jnp.min(cand, axis=1, keepdims=True)

        @pl.when(b == 0)
        def _():
            out_ref[...] = jnp.zeros((S, B), jnp.int32)

        lane = jax.lax.broadcasted_iota(jnp.int32, (S, B), 1)
        out_ref[...] = jnp.where(lane == b, idx, out_ref[...])


def _onehot_kernel(s_ref, out_ref):
    j = pl.program_id(0)
    col = jax.lax.broadcasted_iota(jnp.int32, (S, B, VB2), 2) + j * VB2
    out_ref[...] = (col == s_ref[...][:, :, None]).astype(jnp.float32)


@jax.jit
def kernel(x):
    x_p = jnp.pad(x, ((0, 0), (0, VPAD - V)), constant_values=-jnp.inf)
    x_p = x_p.reshape(B * NJ, 1, VB)
    samples = pl.pallas_call(
        _sample_kernel,
        grid=(B, NJ),
        in_specs=[pl.BlockSpec((1, 1, VB), lambda b, j: (b * NJ + j, 0, 0))],
        out_specs=pl.BlockSpec((S, B), lambda b, j: (0, 0)),
        out_shape=jax.ShapeDtypeStruct((S, B), jnp.int32),
        scratch_shapes=[
            pltpu.VMEM((S, VB), jnp.float32),
            pltpu.VMEM((S, VB), jnp.int32),
        ],
    )(x_p)
    out = pl.pallas_call(
        _onehot_kernel,
        grid=(NJ2,),
        in_specs=[pl.BlockSpec((S, B), lambda j: (0, 0))],
        out_specs=pl.BlockSpec((S, B, VB2), lambda j: (0, 0, j)),
        out_shape=jax.ShapeDtypeStruct((S, B, V), jnp.float32),
    )(samples)
    return out
